# Initial kernel scaffold; baseline (speedup 1.0000x reference)
import jax
import jax.numpy as jnp
from jax.experimental import pallas as pl


def kernel(c3_seq, d_seq, r_seq, v_c3, D_table, v_d, R_table, W_ih, W_hh, b_ih, b_hh, W1a, b1a, W1b, b1b, W2a, b2a, W2b, b2b):
    raise NotImplementedError("write your pallas kernel here")


# speedup vs baseline: 31.0533x; 31.0533x over previous
import functools

import jax
import jax.numpy as jnp
from jax import lax
from jax.experimental import pallas as pl
from jax.experimental.pallas import tpu as pltpu
from jax.experimental.pallas import tpu_sc as plsc

B, S = 64, 200
NUM_C3, NUM_D, DIM_V = 2048, 10000, 128
CHUNK = 8
NCHUNK = S // CHUNK
F32 = jnp.float32


def _sc_gather_gamma(table_flat, idx_flat):
    info = plsc.get_sparse_core_info()
    nc, ns, nl = info.num_cores, info.num_subcores, info.num_lanes
    nw = nc * ns
    total = B * S
    per = total // nw
    assert per % nl == 0 and total % nw == 0

    mesh = plsc.VectorSubcoreMesh(core_axis_name="c", subcore_axis_name="s")

    @functools.partial(
        pl.kernel,
        mesh=mesh,
        out_type=jax.ShapeDtypeStruct((total,), F32),
        scratch_types=[
            pltpu.VMEM((NUM_D,), F32),
            pltpu.VMEM((per,), jnp.int32),
            pltpu.VMEM((per,), F32),
        ],
        compiler_params=pltpu.CompilerParams(needs_layout_passes=False),
    )
    def gather_kernel(table_hbm, idx_hbm, out_hbm, tab_v, idx_v, out_v):
        wid = lax.axis_index("s") * nc + lax.axis_index("c")
        base = wid * per
        pltpu.sync_copy(table_hbm, tab_v)
        pltpu.sync_copy(idx_hbm.at[pl.ds(base, per)], idx_v)
        for i in range(per // nl):
            idx = idx_v[pl.ds(i * nl, nl)]
            out_v[pl.ds(i * nl, nl)] = plsc.load_gather(tab_v, [idx])
        pltpu.sync_copy(out_v, out_hbm.at[pl.ds(base, per)])

    return gather_kernel(table_flat, idx_flat)


def _tc_body(gamma_ref, r_ref, c3_ref,
             WihT_ref, WhhT_ref, R_ref, vd_ref, vc3_ref,
             bih_ref, bhh_ref,
             W1aT_ref, w1b_ref, b1a_ref, b1b_ref,
             W2aT_ref, w2b_ref, b2a_ref, b2b_ref,
             alpha_out_ref, h_out_ref, c3_out_ref,
             h_scr, alpha_scr, c3_scr):
    pid = pl.program_id(0)

    @pl.when(pid == 0)
    def _init():
        h_scr[...] = jnp.zeros_like(h_scr)
        alpha_scr[...] = jnp.zeros_like(alpha_scr)
        c3_scr[...] = jnp.zeros_like(c3_scr)

    vd_row = vd_ref[...]
    a_row = jnp.dot(vd_row, WihT_ref[0:DIM_V, :],
                    preferred_element_type=F32)
    c01 = jnp.dot(R_ref[...], WihT_ref[DIM_V:2 * DIM_V, :],
                  preferred_element_type=F32)
    u1_row = jnp.dot(vc3_ref[...], W2aT_ref[0:DIM_V, :],
                     preferred_element_type=F32)
    ud_row = jnp.dot(vd_row, W2aT_ref[DIM_V:2 * DIM_V, :],
                     preferred_element_type=F32)
    c2 = jnp.dot(R_ref[...], W2aT_ref[2 * DIM_V:3 * DIM_V, :],
                 preferred_element_type=F32)

    bih_row = bih_ref[...] + bhh_ref[...]
    b1a_row = b1a_ref[...]
    b2a_row = b2a_ref[...]
    w1b_row = w1b_ref[...]
    w2b_row = w2b_ref[...]
    b1b = b1b_ref[...]
    b2b = b2b_ref[...]
    WhhT = WhhT_ref[...]
    W1aT = W1aT_ref[...]

    iota = lax.broadcasted_iota(jnp.int32, (B, NUM_C3), 1)

    h = h_scr[...]
    alpha = alpha_scr[...]
    c3 = c3_scr[...]

    for k in range(CHUNK):
        g_col = gamma_ref[0, :, k:k + 1]
        r_col = r_ref[0, :, k:k + 1]
        c_col = c3_ref[0, :, k:k + 1]
        r_is1 = r_col == 1

        gi = g_col * a_row + jnp.where(r_is1, c01[1:2, :], c01[0:1, :]) \
            + bih_row
        gh = jnp.dot(h, WhhT, preferred_element_type=F32)
        rg = jax.nn.sigmoid(gi[:, 0:DIM_V] + gh[:, 0:DIM_V])
        zg = jax.nn.sigmoid(gi[:, DIM_V:2 * DIM_V] + gh[:, DIM_V:2 * DIM_V])
        ng = jnp.tanh(gi[:, 2 * DIM_V:] + rg * gh[:, 2 * DIM_V:])
        h = (1.0 - zg) * ng + zg * h
        h_out_ref[:, k, :] = h

        hid = jax.nn.relu(jnp.dot(h, W1aT, preferred_element_type=F32)
                          + b1a_row)
        alpha_new = jnp.sum(hid * w1b_row, axis=1, keepdims=True) + b1b
        cond = (alpha - g_col) >= 0.0
        take_new = jnp.logical_not(jnp.logical_xor(r_is1, cond))
        alpha = jnp.where(take_new, alpha_new, alpha)
        alpha_out_ref[0, :, k:k + 1] = alpha

        mask = iota == c_col
        beta3 = jnp.sum(jnp.where(mask, c3, 0.0), axis=1, keepdims=True)
        pre2 = beta3 * u1_row + g_col * ud_row \
            + jnp.where(r_is1, c2[1:2, :], c2[0:1, :]) + b2a_row
        new_c3 = jnp.sum(jax.nn.relu(pre2) * w2b_row, axis=1,
                         keepdims=True) + b2b
        c3 = jnp.where(mask, new_c3, c3)
        c3_out_ref[:, k, :] = c3

    h_scr[...] = h
    alpha_scr[...] = alpha
    c3_scr[...] = c3


def _run_tc(gamma_c, r_c, c3_c, WihT, WhhT, R_table, vd_row, vc3_row,
            bih_row, bhh_row, W1aT, w1b, b1a_row, b1b_2d,
            W2aT, w2b, b2a_row, b2b_2d, interpret=False):
    chunk_spec = pl.BlockSpec((1, B, CHUNK), lambda i: (i, 0, 0))

    def full(shape):
        nd = len(shape)
        return pl.BlockSpec(shape, lambda i, _n=nd: (0,) * _n)

    out_shapes = (
        jax.ShapeDtypeStruct((NCHUNK, B, CHUNK), F32),
        jax.ShapeDtypeStruct((B, S, DIM_V), F32),
        jax.ShapeDtypeStruct((B, S, NUM_C3), F32),
    )
    out_specs = (
        chunk_spec,
        pl.BlockSpec((B, CHUNK, DIM_V), lambda i: (0, i, 0)),
        pl.BlockSpec((B, CHUNK, NUM_C3), lambda i: (0, i, 0)),
    )
    in_specs = [
        chunk_spec, chunk_spec, chunk_spec,
        full(WihT.shape), full(WhhT.shape), full(R_table.shape),
        full(vd_row.shape), full(vc3_row.shape),
        full(bih_row.shape), full(bhh_row.shape),
        full(W1aT.shape), full(w1b.shape), full(b1a_row.shape),
        full(b1b_2d.shape),
        full(W2aT.shape), full(w2b.shape), full(b2a_row.shape),
        full(b2b_2d.shape),
    ]
    return pl.pallas_call(
        _tc_body,
        grid=(NCHUNK,),
        in_specs=in_specs,
        out_specs=out_specs,
        out_shape=out_shapes,
        scratch_shapes=[
            pltpu.VMEM((B, DIM_V), F32),
            pltpu.VMEM((B, 1), F32),
            pltpu.VMEM((B, NUM_C3), F32),
        ],
        compiler_params=pltpu.CompilerParams(
            dimension_semantics=("arbitrary",),
        ),
        interpret=interpret,
    )(gamma_c, r_c, c3_c, WihT, WhhT, R_table, vd_row, vc3_row,
      bih_row, bhh_row, W1aT, w1b, b1a_row, b1b_2d,
      W2aT, w2b, b2a_row, b2b_2d)


def _chunked(x):
    return x.reshape(B, NCHUNK, CHUNK).transpose(1, 0, 2)


def kernel(c3_seq, d_seq, r_seq, v_c3, D_table, v_d, R_table, W_ih, W_hh,
           b_ih, b_hh, W1a, b1a, W1b, b1b, W2a, b2a, W2b, b2b):
    gamma_flat = _sc_gather_gamma(
        D_table.reshape(-1).astype(F32),
        d_seq.reshape(-1).astype(jnp.int32),
    )
    gamma = gamma_flat.reshape(B, S)

    alpha_c, h_seq, c3_out = _run_tc(
        _chunked(gamma),
        _chunked(r_seq.astype(jnp.int32)),
        _chunked(c3_seq.astype(jnp.int32)),
        W_ih.T, W_hh.T, R_table,
        v_d.reshape(1, DIM_V), v_c3.reshape(1, DIM_V),
        b_ih.reshape(1, 3 * DIM_V), b_hh.reshape(1, 3 * DIM_V),
        W1a.T, W1b, b1a.reshape(1, DIM_V), b1b.reshape(1, 1),
        W2a.T, W2b, b2a.reshape(1, DIM_V), b2b.reshape(1, 1),
    )
    alpha_seq = alpha_c.transpose(1, 0, 2).reshape(B, S)
    return alpha_seq, h_seq, c3_out
